# baseline (speedup 1.0000x reference)
import functools

import jax
import jax.numpy as jnp
from jax import lax
from jax.experimental import pallas as pl
from jax.experimental.pallas import tpu as pltpu
from jax.experimental.pallas import tpu_sc as plsc

NC = 2
NS = 16
NW = NC * NS
LANES = 16
LPAD = 64


def _make_sc_call(B, L, V, D):
  assert D == 64 and B % NW == 0
  bpw = B // NW
  chunk_items = 2
  rows = chunk_items * LPAD
  nchunk = bpw // chunk_items
  kd = D // LANES

  mesh = plsc.VectorSubcoreMesh(core_axis_name="c", subcore_axis_name="s")

  @functools.partial(
      pl.kernel,
      out_type=jax.ShapeDtypeStruct((B * LPAD,), jnp.float32),
      mesh=mesh,
      compiler_params=pltpu.CompilerParams(
          needs_layout_passes=False, use_tc_tiling_on_sc=False),
      scratch_types=[
          pltpu.VMEM((bpw,), jnp.int32),
          pltpu.VMEM((bpw * LPAD,), jnp.int32),
          pltpu.VMEM((bpw, D), jnp.float32),
          pltpu.VMEM((2, rows, D), jnp.float32),
          pltpu.VMEM((bpw * LPAD,), jnp.float32),
          pltpu.VMEM((LANES * LANES,), jnp.float32),
          pltpu.SemaphoreType.DMA,
          pltpu.SemaphoreType.DMA,
          pltpu.SemaphoreType.DMA,
      ],
  )
  def sc_call(center_hbm, ctx_hbm, ev_hbm, eu_hbm, out_hbm,
              cidx_v, ctxidx_v, vrows_v, ubuf_v, outbuf_v, tr_v,
              sem0, sem1, semv):
    wid = lax.axis_index("s") * NC + lax.axis_index("c")
    item0 = wid * bpw

    pltpu.sync_copy(center_hbm.at[pl.ds(item0 * 1, bpw)], cidx_v)
    pltpu.sync_copy(ctx_hbm.at[pl.ds(item0 * LPAD, bpw * LPAD)], ctxidx_v)

    for q in range(bpw // 128):
      pltpu.async_copy(
          ev_hbm.at[cidx_v.at[pl.ds(q * 128, 128)]],
          vrows_v.at[pl.ds(q * 128, 128)],
          semv,
      ).wait()

    col_base = lax.iota(jnp.int32, LANES) * LANES

    def u_gather(c, buf, sem):
      return pltpu.make_async_copy(
          eu_hbm.at[ctxidx_v.at[pl.ds(c * rows, rows)]], ubuf_v.at[buf], sem)

    u_gather(0, 0, sem0).start()
    u_gather(1, 1, sem1).start()

    def compute_chunk(c, ub):
      for i in range(chunk_items):
        item = c * chunk_items + i
        vvec = [vrows_v[item, pl.ds(k * LANES, LANES)] for k in range(kd)]
        for g in range(LPAD // LANES):
          base = i * LPAD + g * LANES
          for j in range(LANES):
            p = ub[base + j, pl.ds(0, LANES)] * vvec[0]
            for k in range(1, kd):
              p = p + ub[base + j, pl.ds(k * LANES, LANES)] * vvec[k]
            plsc.store_scatter(tr_v, [col_base + j], p)
          acc = tr_v[pl.ds(0, LANES)]
          for r in range(1, LANES):
            acc = acc + tr_v[pl.ds(r * LANES, LANES)]
          outbuf_v[pl.ds(item * LPAD + g * LANES, LANES)] = acc

    def pair_body(gidx, carry):
      for b in range(2):
        c = gidx * 2 + b
        sem = sem0 if b == 0 else sem1
        u_gather(c, b, sem).wait()
        compute_chunk(c, ubuf_v.at[b])

        @pl.when(c + 2 < nchunk)
        def _():
          u_gather(c + 2, b, sem).start()

      return carry

    lax.fori_loop(0, nchunk // 2, pair_body, 0)

    pltpu.sync_copy(outbuf_v, out_hbm.at[pl.ds(item0 * LPAD, bpw * LPAD)])

  return sc_call


def kernel(center, context_negative, embed_v, embed_u):
  B, L = context_negative.shape
  V, D = embed_u.shape
  ctx_pad = jnp.pad(context_negative, ((0, 0), (0, LPAD - L))).reshape(-1)
  sc_call = _make_sc_call(B, L, V, D)
  out = sc_call(center, ctx_pad, embed_v, embed_u)
  return out.reshape(B, LPAD)[:, :L]


# speedup vs baseline: 1.8002x; 1.8002x over previous
import functools

import jax
import jax.numpy as jnp
from jax import lax
from jax.experimental import pallas as pl
from jax.experimental.pallas import tpu as pltpu
from jax.experimental.pallas import tpu_sc as plsc

NC = 2
NS = 16
NW = NC * NS
LANES = 16
LPAD = 56


def _make_sc_call(B, L, V, D):
  assert D == 64 and B % NW == 0
  bpw = B // NW
  chunk_items = 8
  rows = chunk_items * LPAD
  nchunk = bpw // chunk_items
  kd = D // LANES
  full_groups = L // LANES
  tail = L - full_groups * LANES

  mesh = plsc.VectorSubcoreMesh(core_axis_name="c", subcore_axis_name="s")

  @functools.partial(
      pl.kernel,
      out_type=jax.ShapeDtypeStruct((B * L,), jnp.float32),
      mesh=mesh,
      compiler_params=pltpu.CompilerParams(
          needs_layout_passes=False, use_tc_tiling_on_sc=False),
      scratch_types=[
          pltpu.VMEM((bpw,), jnp.int32),
          pltpu.VMEM((2, rows), jnp.int32),
          pltpu.VMEM((bpw, D), jnp.float32),
          pltpu.VMEM((2, rows, D), jnp.float32),
          pltpu.VMEM((bpw * L,), jnp.float32),
          pltpu.VMEM((LANES * LANES,), jnp.float32),
          pltpu.SemaphoreType.DMA,
          pltpu.SemaphoreType.DMA,
          pltpu.SemaphoreType.DMA,
          pltpu.SemaphoreType.DMA,
          pltpu.SemaphoreType.DMA,
      ],
  )
  def sc_call(center_hbm, ctx_hbm, ev_hbm, eu_hbm, out_hbm,
              cidx_v, ctxidx_v, vrows_v, ubuf_v, outbuf_v, tr_v,
              semg0, semg1, semi0, semi1, semv):
    wid = lax.axis_index("s") * NC + lax.axis_index("c")
    item0 = wid * bpw

    pltpu.sync_copy(center_hbm.at[pl.ds(item0, bpw)], cidx_v)
    pltpu.async_copy(ev_hbm.at[cidx_v], vrows_v, semv).wait()

    def idx_stage(c, buf, sem):
      return pltpu.make_async_copy(
          ctx_hbm.at[pl.ds(item0 * LPAD + c * rows, rows)],
          ctxidx_v.at[buf], sem)

    def u_gather(buf, sem):
      return pltpu.make_async_copy(
          eu_hbm.at[ctxidx_v.at[buf]], ubuf_v.at[buf], sem)

    idx_stage(0, 0, semi0).start()
    idx_stage(1, 1, semi1).start()
    idx_stage(0, 0, semi0).wait()
    u_gather(0, semg0).start()
    idx_stage(1, 1, semi1).wait()
    u_gather(1, semg1).start()

    col_base = lax.iota(jnp.int32, LANES) * LANES
    tail_mask = lax.iota(jnp.int32, LANES) < tail

    def compute_item(il, c, parity):
      item = c * chunk_items + il
      ub_row0 = il * LPAD
      vvec = [vrows_v[item, pl.ds(k * LANES, LANES)] for k in range(kd)]

      def partials(base, nj):
        for j in range(nj):
          p = ubuf_v[parity, base + j, pl.ds(0, LANES)] * vvec[0]
          for k in range(1, kd):
            p = p + ubuf_v[parity, base + j, pl.ds(k * LANES, LANES)] * vvec[k]
          plsc.store_scatter(tr_v, [col_base + j], p)

      def rowsum():
        acc = tr_v[pl.ds(0, LANES)]
        for r in range(1, LANES):
          acc = acc + tr_v[pl.ds(r * LANES, LANES)]
        return acc

      for g in range(full_groups):
        partials(ub_row0 + g * LANES, LANES)
        outbuf_v[pl.ds(item * L + g * LANES, LANES)] = rowsum()
      partials(ub_row0 + full_groups * LANES, 8)
      plsc.store_compressed(
          outbuf_v.at[pl.ds(item * L + full_groups * LANES, LANES)],
          rowsum(), mask=tail_mask)

    def chunk_body(c, parity, semg, semi):
      u_gather(parity, semg).wait()

      @pl.when(c + 2 < nchunk)
      def _():
        idx_stage(c + 2, parity, semi).start()

      lax.fori_loop(
          0, chunk_items, lambda il, _: (compute_item(il, c, parity), 0)[1], 0)

      @pl.when(c + 2 < nchunk)
      def _():
        idx_stage(c + 2, parity, semi).wait()
        u_gather(parity, semg).start()

    def pair_body(gidx, carry):
      chunk_body(gidx * 2, 0, semg0, semi0)
      chunk_body(gidx * 2 + 1, 1, semg1, semi1)
      return carry

    lax.fori_loop(0, nchunk // 2, pair_body, 0)

    pltpu.sync_copy(outbuf_v, out_hbm.at[pl.ds(item0 * L, bpw * L)])

  return sc_call


def kernel(center, context_negative, embed_v, embed_u):
  B, L = context_negative.shape
  V, D = embed_u.shape
  ctx_pad = jnp.pad(context_negative, ((0, 0), (0, LPAD - L))).reshape(-1)
  sc_call = _make_sc_call(B, L, V, D)
  out = sc_call(center, ctx_pad, embed_v, embed_u)
  return out.reshape(B, L)


# speedup vs baseline: 4.5188x; 2.5101x over previous
import functools

import jax
import jax.numpy as jnp
from jax import lax
from jax.experimental import pallas as pl
from jax.experimental.pallas import tpu as pltpu
from jax.experimental.pallas import tpu_sc as plsc

NC = 2
NS = 16
NW = NC * NS
LANES = 16


def _make_sc_call(B, L, V, D):
  assert D == 64 and B % NW == 0 and L == 50
  bpw = B // NW
  chunk_items = 8
  rows = chunk_items * L
  nchunk = bpw // chunk_items
  kd = D // LANES
  group_offs = (0, 16, 32, L - LANES)

  mesh = plsc.VectorSubcoreMesh(core_axis_name="c", subcore_axis_name="s")

  @functools.partial(
      pl.kernel,
      out_type=jax.ShapeDtypeStruct((B, L), jnp.float32),
      mesh=mesh,
      compiler_params=pltpu.CompilerParams(
          needs_layout_passes=False, use_tc_tiling_on_sc=False),
      scratch_types=[
          pltpu.VMEM((bpw,), jnp.int32),
          pltpu.VMEM((2, chunk_items, L), jnp.int32),
          pltpu.VMEM((2 * rows,), jnp.int32),
          pltpu.VMEM((bpw, D), jnp.float32),
          pltpu.VMEM((2, rows, D), jnp.float32),
          pltpu.VMEM((bpw, L), jnp.float32),
          pltpu.SemaphoreType.DMA,
          pltpu.SemaphoreType.DMA,
          pltpu.SemaphoreType.DMA,
          pltpu.SemaphoreType.DMA,
          pltpu.SemaphoreType.DMA,
      ],
  )
  def sc_call(center_hbm, ctx_hbm, ev_hbm, eu_hbm, out_hbm,
              cidx_v, rawidx_v, flatidx_v, vrows_v, ubuf_v, outbuf_v,
              semg0, semg1, semi0, semi1, semv):
    wid = lax.axis_index("s") * NC + lax.axis_index("c")
    item0 = wid * bpw

    pltpu.sync_copy(center_hbm.at[pl.ds(item0, bpw)], cidx_v)
    pltpu.async_copy(ev_hbm.at[cidx_v], vrows_v, semv).wait()

    lane_iota = lax.iota(jnp.int32, LANES)

    def idx_stage(c, buf, sem):
      return pltpu.make_async_copy(
          ctx_hbm.at[pl.ds(item0 + c * chunk_items, chunk_items), :],
          rawidx_v.at[buf], sem)

    def flatten_idx(parity):
      for il in range(chunk_items):
        for off in group_offs:
          flatidx_v[pl.ds(parity * rows + il * L + off, LANES)] = (
              rawidx_v[parity, il, pl.ds(off, LANES)])

    def u_gather(buf, sem):
      return pltpu.make_async_copy(
          eu_hbm.at[flatidx_v.at[pl.ds(buf * rows, rows)]],
          ubuf_v.at[buf], sem)

    idx_stage(0, 0, semi0).start()
    idx_stage(1, 1, semi1).start()
    idx_stage(0, 0, semi0).wait()
    flatten_idx(0)
    u_gather(0, semg0).start()
    idx_stage(1, 1, semi1).wait()
    flatten_idx(1)
    u_gather(1, semg1).start()

    def compute_item(il, c, parity):
      item = c * chunk_items + il
      ub_row0 = il * L
      vvec = [vrows_v[item, pl.ds(k * LANES, LANES)] for k in range(kd)]

      def group(base):
        acc = jnp.full((LANES,), 0.0, jnp.float32)
        for j in range(LANES):
          p = ubuf_v[parity, base + j, pl.ds(0, LANES)] * vvec[0]
          for k in range(1, kd):
            p = p + ubuf_v[parity, base + j, pl.ds(k * LANES, LANES)] * vvec[k]
          acc = jnp.where(lane_iota == j, lax.reduce_sum(p, axes=(0,)), acc)
        return acc

      for off in group_offs:
        outbuf_v[item, pl.ds(off, LANES)] = group(ub_row0 + off)

    def chunk_body(c, parity, semg, semi):
      u_gather(parity, semg).wait()

      @pl.when(c + 2 < nchunk)
      def _():
        idx_stage(c + 2, parity, semi).start()

      lax.fori_loop(
          0, chunk_items, lambda il, _: (compute_item(il, c, parity), 0)[1], 0)

      @pl.when(c + 2 < nchunk)
      def _():
        idx_stage(c + 2, parity, semi).wait()
        flatten_idx(parity)
        u_gather(parity, semg).start()

    def pair_body(gidx, carry):
      chunk_body(gidx * 2, 0, semg0, semi0)
      chunk_body(gidx * 2 + 1, 1, semg1, semi1)
      return carry

    lax.fori_loop(0, nchunk // 2, pair_body, 0)

    pltpu.sync_copy(outbuf_v, out_hbm.at[pl.ds(item0, bpw), :])

  return sc_call


def kernel(center, context_negative, embed_v, embed_u):
  B, L = context_negative.shape
  V, D = embed_u.shape
  sc_call = _make_sc_call(B, L, V, D)
  return sc_call(center, context_negative, embed_v, embed_u)
